# Initial kernel scaffold; baseline (speedup 1.0000x reference)
import jax
import jax.numpy as jnp
from jax.experimental import pallas as pl


def kernel(student_output, teacher_output, partition_size):
    raise NotImplementedError("write your pallas kernel here")


# speedup vs baseline: 2.3866x; 2.3866x over previous
import functools

import numpy as np
import jax
import jax.numpy as jnp
from jax import lax
from jax.experimental import pallas as pl
from jax.experimental.pallas import tpu as pltpu
from jax.experimental.pallas import tpu_sc as plsc

_NPROTO = 65536
_PSIZE = 128
_NPART = _NPROTO // _PSIZE
_NCROPS = 10
_SB = 640
_TB = 128


def _threefry2x32(k0, k1, x0, x1):
    x0 = x0.astype(np.uint32).copy()
    x1 = x1.astype(np.uint32).copy()
    ks = [np.uint32(k0), np.uint32(k1),
          np.uint32(k0) ^ np.uint32(k1) ^ np.uint32(0x1BD11BDA)]
    rotations = [[13, 15, 26, 6], [17, 29, 16, 24]]
    x0 = (x0 + ks[0]).astype(np.uint32)
    x1 = (x1 + ks[1]).astype(np.uint32)
    for i in range(5):
        for r in rotations[i % 2]:
            x0 = (x0 + x1).astype(np.uint32)
            x1 = ((x1 << np.uint32(r)) | (x1 >> np.uint32(32 - r))).astype(np.uint32)
            x1 = (x0 ^ x1).astype(np.uint32)
        x0 = (x0 + ks[(i + 1) % 3]).astype(np.uint32)
        x1 = (x1 + ks[(i + 2) % 3] + np.uint32(i + 1)).astype(np.uint32)
    return x0, x1


def _random_bits(k0, k1, n):
    hi = np.zeros(n, dtype=np.uint32)
    lo = np.arange(n, dtype=np.uint32)
    o0, o1 = _threefry2x32(k0, k1, hi, lo)
    return o0 ^ o1


def _split_key(k0, k1):
    hi = np.zeros(2, dtype=np.uint32)
    lo = np.arange(2, dtype=np.uint32)
    o0, o1 = _threefry2x32(k0, k1, hi, lo)
    return np.stack([o0, o1], axis=1)


def _perm_rows() -> np.ndarray:
    k = (np.uint32(0), np.uint32(42))
    x = np.arange(_NPROTO, dtype=np.int32)
    for _ in range(2):
        ks = _split_key(*k)
        k = (ks[0, 0], ks[0, 1])
        sort_keys = _random_bits(ks[1, 0], ks[1, 1], _NPROTO)
        x = x[np.argsort(sort_keys, kind="stable")]
    return x.reshape(_NPART, _PSIZE)


_PERM2D = _perm_rows()


def _transpose_body(s_ref, t_ref, ts_ref, tt_ref):
    ts_ref[...] = s_ref[...].T
    tt_ref[...] = t_ref[...].T


def _transpose(student, teacher):
    jb = 2048
    return pl.pallas_call(
        _transpose_body,
        grid=(_NPROTO // jb,),
        in_specs=[
            pl.BlockSpec((_SB, jb), lambda j: (0, j)),
            pl.BlockSpec((_TB, jb), lambda j: (0, j)),
        ],
        out_specs=[
            pl.BlockSpec((jb, _SB), lambda j: (j, 0)),
            pl.BlockSpec((jb, _TB), lambda j: (j, 0)),
        ],
        out_shape=[
            jax.ShapeDtypeStruct((_NPROTO, _SB), jnp.float32),
            jax.ShapeDtypeStruct((_NPROTO, _TB), jnp.float32),
        ],
    )(student, teacher)


def _sc_gather(ts, tt, perm2d):
    info = plsc.get_sparse_core_info()
    nc, ns = info.num_cores, info.num_subcores
    nw = nc * ns
    rows_per_w = _NPROTO // nw
    chunks = rows_per_w // _PSIZE

    mesh = plsc.VectorSubcoreMesh(core_axis_name="c", subcore_axis_name="s")

    @functools.partial(
        pl.kernel,
        mesh=mesh,
        out_type=[
            jax.ShapeDtypeStruct((_NPROTO, _SB), jnp.float32),
            jax.ShapeDtypeStruct((_NPROTO, _TB), jnp.float32),
        ],
        scratch_types=[
            pltpu.VMEM((chunks, _PSIZE), jnp.int32),
            pltpu.VMEM((_PSIZE, _SB), jnp.float32),
            pltpu.VMEM((_PSIZE, _TB), jnp.float32),
            pltpu.SemaphoreType.DMA,
            pltpu.SemaphoreType.DMA,
        ],
    )
    def gather_k(ts_hbm, tt_hbm, perm_hbm, gs_hbm, gt_hbm,
                 idx_v, buf_s, buf_t, sem_s, sem_t):
        wid = lax.axis_index("s") * nc + lax.axis_index("c")
        row0 = wid * rows_per_w
        pltpu.sync_copy(perm_hbm.at[pl.ds(wid * chunks, chunks)], idx_v)

        def body(j, carry):
            idxr = idx_v.at[j]
            cp_s = pltpu.async_copy(ts_hbm.at[idxr], buf_s, sem_s)
            cp_t = pltpu.async_copy(tt_hbm.at[idxr], buf_t, sem_t)
            cp_s.wait()
            cp_t.wait()
            base = row0 + j * _PSIZE
            pltpu.sync_copy(buf_s, gs_hbm.at[pl.ds(base, _PSIZE)])
            pltpu.sync_copy(buf_t, gt_hbm.at[pl.ds(base, _PSIZE)])
            return carry

        lax.fori_loop(0, chunks, body, 0)

    return gather_k(ts, tt, perm2d)


def _softmax_body(gs_ref, gt_ref, p_ref, t_ref):
    x = gs_ref[...]
    x = x - jnp.max(x, axis=0, keepdims=True)
    e = jnp.exp(x)
    r = e / jnp.sum(e, axis=0, keepdims=True)
    p_ref[...] = r.T.reshape(_NCROPS, 1, 64, _PSIZE)

    y = gt_ref[...]
    y = y - jnp.max(y, axis=0, keepdims=True)
    f = jnp.exp(y)
    q = f / jnp.sum(f, axis=0, keepdims=True)
    t_ref[...] = q.T.reshape(2, 1, 64, _PSIZE)


def _softmax(gs, gt):
    return pl.pallas_call(
        _softmax_body,
        grid=(_NPART,),
        in_specs=[
            pl.BlockSpec((_PSIZE, _SB), lambda p: (p, 0)),
            pl.BlockSpec((_PSIZE, _TB), lambda p: (p, 0)),
        ],
        out_specs=[
            pl.BlockSpec((_NCROPS, 1, 64, _PSIZE), lambda p: (0, p, 0, 0)),
            pl.BlockSpec((2, 1, 64, _PSIZE), lambda p: (0, p, 0, 0)),
        ],
        out_shape=[
            jax.ShapeDtypeStruct((_NCROPS, _NPART, 64, _PSIZE), jnp.float32),
            jax.ShapeDtypeStruct((2, _NPART, 64, _PSIZE), jnp.float32),
        ],
    )(gs, gt)


def kernel(student_output, teacher_output, partition_size):
    del partition_size
    perm2d = jnp.asarray(_PERM2D)
    ts, tt = _transpose(student_output, teacher_output)
    gs, gt = _sc_gather(ts, tt, perm2d)
    probs, targets = _softmax(gs, gt)
    return probs, targets


# speedup vs baseline: 3.4629x; 1.4510x over previous
import functools

import numpy as np
import jax
import jax.numpy as jnp
from jax import lax
from jax.experimental import pallas as pl
from jax.experimental.pallas import tpu as pltpu
from jax.experimental.pallas import tpu_sc as plsc

_NPROTO = 65536
_PSIZE = 128
_NPART = _NPROTO // _PSIZE
_NCROPS = 10
_SB = 640
_TB = 128


def _threefry2x32(k0, k1, x0, x1):
    x0 = x0.astype(np.uint32).copy()
    x1 = x1.astype(np.uint32).copy()
    ks = [np.uint32(k0), np.uint32(k1),
          np.uint32(k0) ^ np.uint32(k1) ^ np.uint32(0x1BD11BDA)]
    rotations = [[13, 15, 26, 6], [17, 29, 16, 24]]
    x0 = (x0 + ks[0]).astype(np.uint32)
    x1 = (x1 + ks[1]).astype(np.uint32)
    for i in range(5):
        for r in rotations[i % 2]:
            x0 = (x0 + x1).astype(np.uint32)
            x1 = ((x1 << np.uint32(r)) | (x1 >> np.uint32(32 - r))).astype(np.uint32)
            x1 = (x0 ^ x1).astype(np.uint32)
        x0 = (x0 + ks[(i + 1) % 3]).astype(np.uint32)
        x1 = (x1 + ks[(i + 2) % 3] + np.uint32(i + 1)).astype(np.uint32)
    return x0, x1


def _random_bits(k0, k1, n):
    hi = np.zeros(n, dtype=np.uint32)
    lo = np.arange(n, dtype=np.uint32)
    o0, o1 = _threefry2x32(k0, k1, hi, lo)
    return o0 ^ o1


def _split_key(k0, k1):
    hi = np.zeros(2, dtype=np.uint32)
    lo = np.arange(2, dtype=np.uint32)
    o0, o1 = _threefry2x32(k0, k1, hi, lo)
    return np.stack([o0, o1], axis=1)


def _perm_rows() -> np.ndarray:
    k = (np.uint32(0), np.uint32(42))
    x = np.arange(_NPROTO, dtype=np.int32)
    for _ in range(2):
        ks = _split_key(*k)
        k = (ks[0, 0], ks[0, 1])
        sort_keys = _random_bits(ks[1, 0], ks[1, 1], _NPROTO)
        x = x[np.argsort(sort_keys, kind="stable")]
    return x.reshape(_NPART, _PSIZE)


_PERM2D = _perm_rows()


def _transpose_body(s_ref, t_ref, ts_ref, tt_ref):
    ts_ref[...] = s_ref[...].T
    tt_ref[...] = t_ref[...].T


def _transpose(student, teacher):
    jb = 2048
    return pl.pallas_call(
        _transpose_body,
        grid=(_NPROTO // jb,),
        in_specs=[
            pl.BlockSpec((_SB, jb), lambda j: (0, j)),
            pl.BlockSpec((_TB, jb), lambda j: (0, j)),
        ],
        out_specs=[
            pl.BlockSpec((jb, _SB), lambda j: (j, 0)),
            pl.BlockSpec((jb, _TB), lambda j: (j, 0)),
        ],
        out_shape=[
            jax.ShapeDtypeStruct((_NPROTO, _SB), jnp.float32),
            jax.ShapeDtypeStruct((_NPROTO, _TB), jnp.float32),
        ],
    )(student, teacher)


def _sc_gather(ts, tt, perm2d):
    info = plsc.get_sparse_core_info()
    nc, ns = info.num_cores, info.num_subcores
    nw = nc * ns
    rows_per_w = _NPROTO // nw
    cr = 64
    chunks = rows_per_w // cr

    mesh = plsc.VectorSubcoreMesh(core_axis_name="c", subcore_axis_name="s")

    @functools.partial(
        pl.kernel,
        mesh=mesh,
        out_type=[
            jax.ShapeDtypeStruct((_NPROTO, _SB), jnp.float32),
            jax.ShapeDtypeStruct((_NPROTO, _TB), jnp.float32),
        ],
        scratch_types=[
            pltpu.VMEM((chunks, cr), jnp.int32),
            pltpu.VMEM((cr, _SB), jnp.float32),
            pltpu.VMEM((cr, _SB), jnp.float32),
            pltpu.VMEM((cr, _TB), jnp.float32),
            pltpu.VMEM((cr, _TB), jnp.float32),
            pltpu.SemaphoreType.DMA,
            pltpu.SemaphoreType.DMA,
            pltpu.SemaphoreType.DMA,
            pltpu.SemaphoreType.DMA,
        ],
    )
    def gather_k(ts_hbm, tt_hbm, perm_hbm, gs_hbm, gt_hbm,
                 idx_v, buf_s0, buf_s1, buf_t0, buf_t1,
                 sem_s0, sem_s1, sem_t0, sem_t1):
        wid = lax.axis_index("s") * nc + lax.axis_index("c")
        row0 = wid * rows_per_w
        pltpu.sync_copy(perm_hbm.at[pl.ds(wid * chunks, chunks)], idx_v)
        bufs_s = (buf_s0, buf_s1)
        bufs_t = (buf_t0, buf_t1)
        sems_s = (sem_s0, sem_s1)
        sems_t = (sem_t0, sem_t1)

        def start(j):
            slot = j % 2
            idxr = idx_v.at[j]
            return (pltpu.async_copy(ts_hbm.at[idxr], bufs_s[slot], sems_s[slot]),
                    pltpu.async_copy(tt_hbm.at[idxr], bufs_t[slot], sems_t[slot]))

        pending = start(0)
        for j in range(chunks):
            slot = j % 2
            nxt = start(j + 1) if j + 1 < chunks else None
            pending[0].wait()
            pending[1].wait()
            base = row0 + j * cr
            pltpu.sync_copy(bufs_s[slot], gs_hbm.at[pl.ds(base, cr)])
            pltpu.sync_copy(bufs_t[slot], gt_hbm.at[pl.ds(base, cr)])
            pending = nxt

    return gather_k(ts, tt, perm2d)


_PB = 4


def _softmax_body(gs_ref, gt_ref, p_ref, t_ref):
    x = gs_ref[...].reshape(_PB, _PSIZE, _SB)
    x = x - jnp.max(x, axis=1, keepdims=True)
    e = jnp.exp(x)
    r = e / jnp.sum(e, axis=1, keepdims=True)
    rt = jnp.transpose(r, (0, 2, 1))
    rt = rt.reshape(_PB, _NCROPS, 64, _PSIZE)
    p_ref[...] = jnp.transpose(rt, (1, 0, 2, 3))

    y = gt_ref[...].reshape(_PB, _PSIZE, _TB)
    y = y - jnp.max(y, axis=1, keepdims=True)
    f = jnp.exp(y)
    q = f / jnp.sum(f, axis=1, keepdims=True)
    qt = jnp.transpose(q, (0, 2, 1)).reshape(_PB, 2, 64, _PSIZE)
    t_ref[...] = jnp.transpose(qt, (1, 0, 2, 3))


def _softmax(gs, gt):
    return pl.pallas_call(
        _softmax_body,
        grid=(_NPART // _PB,),
        in_specs=[
            pl.BlockSpec((_PB * _PSIZE, _SB), lambda p: (p, 0)),
            pl.BlockSpec((_PB * _PSIZE, _TB), lambda p: (p, 0)),
        ],
        out_specs=[
            pl.BlockSpec((_NCROPS, _PB, 64, _PSIZE), lambda p: (0, p, 0, 0)),
            pl.BlockSpec((2, _PB, 64, _PSIZE), lambda p: (0, p, 0, 0)),
        ],
        out_shape=[
            jax.ShapeDtypeStruct((_NCROPS, _NPART, 64, _PSIZE), jnp.float32),
            jax.ShapeDtypeStruct((2, _NPART, 64, _PSIZE), jnp.float32),
        ],
    )(gs, gt)


def kernel(student_output, teacher_output, partition_size):
    del partition_size
    perm2d = jnp.asarray(_PERM2D.reshape(_NPROTO // 64, 64))
    ts, tt = _transpose(student_output, teacher_output)
    gs, gt = _sc_gather(ts, tt, perm2d)
    probs, targets = _softmax(gs, gt)
    return probs, targets


# speedup vs baseline: 3.7740x; 1.0898x over previous
import functools

import numpy as np
import jax
import jax.numpy as jnp
from jax import lax
from jax.experimental import pallas as pl
from jax.experimental.pallas import tpu as pltpu
from jax.experimental.pallas import tpu_sc as plsc

_NPROTO = 65536
_PSIZE = 128
_NPART = _NPROTO // _PSIZE
_NCROPS = 10
_SB = 640
_TB = 128


def _threefry2x32(k0, k1, x0, x1):
    x0 = x0.astype(np.uint32).copy()
    x1 = x1.astype(np.uint32).copy()
    ks = [np.uint32(k0), np.uint32(k1),
          np.uint32(k0) ^ np.uint32(k1) ^ np.uint32(0x1BD11BDA)]
    rotations = [[13, 15, 26, 6], [17, 29, 16, 24]]
    x0 = (x0 + ks[0]).astype(np.uint32)
    x1 = (x1 + ks[1]).astype(np.uint32)
    for i in range(5):
        for r in rotations[i % 2]:
            x0 = (x0 + x1).astype(np.uint32)
            x1 = ((x1 << np.uint32(r)) | (x1 >> np.uint32(32 - r))).astype(np.uint32)
            x1 = (x0 ^ x1).astype(np.uint32)
        x0 = (x0 + ks[(i + 1) % 3]).astype(np.uint32)
        x1 = (x1 + ks[(i + 2) % 3] + np.uint32(i + 1)).astype(np.uint32)
    return x0, x1


def _random_bits(k0, k1, n):
    hi = np.zeros(n, dtype=np.uint32)
    lo = np.arange(n, dtype=np.uint32)
    o0, o1 = _threefry2x32(k0, k1, hi, lo)
    return o0 ^ o1


def _split_key(k0, k1):
    hi = np.zeros(2, dtype=np.uint32)
    lo = np.arange(2, dtype=np.uint32)
    o0, o1 = _threefry2x32(k0, k1, hi, lo)
    return np.stack([o0, o1], axis=1)


def _perm_rows() -> np.ndarray:
    k = (np.uint32(0), np.uint32(42))
    x = np.arange(_NPROTO, dtype=np.int32)
    for _ in range(2):
        ks = _split_key(*k)
        k = (ks[0, 0], ks[0, 1])
        sort_keys = _random_bits(ks[1, 0], ks[1, 1], _NPROTO)
        x = x[np.argsort(sort_keys, kind="stable")]
    return x.reshape(_NPART, _PSIZE)


_PERM2D = _perm_rows()


def _transpose_body(s_ref, t_ref, ts_ref, tt_ref):
    ts_ref[...] = s_ref[...].T
    tt_ref[...] = t_ref[...].T


def _transpose(student, teacher):
    jb = 2048
    return pl.pallas_call(
        _transpose_body,
        grid=(_NPROTO // jb,),
        in_specs=[
            pl.BlockSpec((_SB, jb), lambda j: (0, j)),
            pl.BlockSpec((_TB, jb), lambda j: (0, j)),
        ],
        out_specs=[
            pl.BlockSpec((jb, _SB), lambda j: (j, 0)),
            pl.BlockSpec((jb, _TB), lambda j: (j, 0)),
        ],
        out_shape=[
            jax.ShapeDtypeStruct((_NPROTO, _SB), jnp.float32),
            jax.ShapeDtypeStruct((_NPROTO, _TB), jnp.float32),
        ],
    )(student, teacher)


def _sc_gather(ts, tt, perm2d):
    info = plsc.get_sparse_core_info()
    nc, ns = info.num_cores, info.num_subcores
    nw = nc * ns
    rows_per_w = _NPROTO // nw
    cr = 64
    chunks = rows_per_w // cr

    mesh = plsc.VectorSubcoreMesh(core_axis_name="c", subcore_axis_name="s")

    @functools.partial(
        pl.kernel,
        mesh=mesh,
        out_type=[
            jax.ShapeDtypeStruct((_NPROTO, _SB), jnp.float32),
            jax.ShapeDtypeStruct((_NPROTO, _TB), jnp.float32),
        ],
        scratch_types=[
            pltpu.VMEM((chunks, cr), jnp.int32),
            pltpu.VMEM((cr, _SB), jnp.float32),
            pltpu.VMEM((cr, _SB), jnp.float32),
            pltpu.VMEM((cr, _TB), jnp.float32),
            pltpu.VMEM((cr, _TB), jnp.float32),
            pltpu.SemaphoreType.DMA,
            pltpu.SemaphoreType.DMA,
            pltpu.SemaphoreType.DMA,
            pltpu.SemaphoreType.DMA,
        ],
    )
    def gather_k(ts_hbm, tt_hbm, perm_hbm, gs_hbm, gt_hbm,
                 idx_v, buf_s0, buf_s1, buf_t0, buf_t1,
                 sem_s0, sem_s1, sem_t0, sem_t1):
        wid = lax.axis_index("s") * nc + lax.axis_index("c")
        row0 = wid * rows_per_w
        pltpu.sync_copy(perm_hbm.at[pl.ds(wid * chunks, chunks)], idx_v)
        bufs_s = (buf_s0, buf_s1)
        bufs_t = (buf_t0, buf_t1)
        sems_s = (sem_s0, sem_s1)
        sems_t = (sem_t0, sem_t1)

        def start(j):
            slot = j % 2
            idxr = idx_v.at[j]
            return (pltpu.async_copy(ts_hbm.at[idxr], bufs_s[slot], sems_s[slot]),
                    pltpu.async_copy(tt_hbm.at[idxr], bufs_t[slot], sems_t[slot]))

        pending = start(0)
        for j in range(chunks):
            slot = j % 2
            nxt = start(j + 1) if j + 1 < chunks else None
            pending[0].wait()
            pending[1].wait()
            base = row0 + j * cr
            pltpu.sync_copy(bufs_s[slot], gs_hbm.at[pl.ds(base, cr)])
            pltpu.sync_copy(bufs_t[slot], gt_hbm.at[pl.ds(base, cr)])
            pending = nxt

    return gather_k(ts, tt, perm2d)


_PB = 8


def _softmax_body(gs_ref, gt_ref, p_ref, t_ref):
    x = gs_ref[...].reshape(_PB, _PSIZE, _SB)
    x = x - jnp.max(x, axis=1, keepdims=True)
    e = jnp.exp(x)
    r = e / jnp.sum(e, axis=1, keepdims=True)
    rt = jnp.transpose(r, (0, 2, 1))
    rt = rt.reshape(_PB, _NCROPS, 64, _PSIZE)
    p_ref[...] = jnp.transpose(rt, (1, 0, 2, 3))

    y = gt_ref[...].reshape(_PB, _PSIZE, _TB)
    y = y - jnp.max(y, axis=1, keepdims=True)
    f = jnp.exp(y)
    q = f / jnp.sum(f, axis=1, keepdims=True)
    qt = jnp.transpose(q, (0, 2, 1)).reshape(_PB, 2, 64, _PSIZE)
    t_ref[...] = jnp.transpose(qt, (1, 0, 2, 3))


def _softmax(gs, gt):
    return pl.pallas_call(
        _softmax_body,
        grid=(_NPART // _PB,),
        in_specs=[
            pl.BlockSpec((_PB * _PSIZE, _SB), lambda p: (p, 0)),
            pl.BlockSpec((_PB * _PSIZE, _TB), lambda p: (p, 0)),
        ],
        out_specs=[
            pl.BlockSpec((_NCROPS, _PB, 64, _PSIZE), lambda p: (0, p, 0, 0)),
            pl.BlockSpec((2, _PB, 64, _PSIZE), lambda p: (0, p, 0, 0)),
        ],
        out_shape=[
            jax.ShapeDtypeStruct((_NCROPS, _NPART, 64, _PSIZE), jnp.float32),
            jax.ShapeDtypeStruct((2, _NPART, 64, _PSIZE), jnp.float32),
        ],
    )(gs, gt)


def kernel(student_output, teacher_output, partition_size):
    del partition_size
    perm2d = jnp.asarray(_PERM2D.reshape(_NPROTO // 64, 64))
    ts, tt = _transpose(student_output, teacher_output)
    gs, gt = _sc_gather(ts, tt, perm2d)
    probs, targets = _softmax(gs, gt)
    return probs, targets
